# baseline (speedup 1.0000x reference)
import functools

import jax
import jax.numpy as jnp
from jax import lax
from jax.experimental import pallas as pl
from jax.experimental.pallas import tpu as pltpu
from jax.experimental.pallas import tpu_sc as plsc

_NUM_WORKERS = 32
_LANES = 16


def _make_mask_kernel(n_words: int, g_pad: int):
    from jax._src.pallas import mpmd as _plmpmd

    per_w = g_pad // _NUM_WORKERS
    n_a = 128
    n_b = per_w - n_a
    mesh = plsc.VectorSubcoreMesh(core_axis_name="c", subcore_axis_name="s")

    def mask_body(g_hbm, zeros_hbm, mask_hbm, idx_a, idx_b, ones_a, ones_b, sem):
        del zeros_hbm
        c = lax.axis_index("c")
        s = lax.axis_index("s")
        wid = s * 2 + c
        base = wid * per_w

        pltpu.sync_copy(g_hbm.at[pl.ds(base, n_a)], idx_a)
        pltpu.sync_copy(g_hbm.at[pl.ds(base + n_a, n_b)], idx_b)

        ones16 = jnp.ones((_LANES,), jnp.float32)
        for j in range(n_a // _LANES):
            ones_a[pl.ds(j * _LANES, _LANES)] = ones16
        for j in range(n_b // _LANES):
            ones_b[pl.ds(j * _LANES, _LANES)] = ones16

        sa = pltpu.async_copy(ones_a, mask_hbm.at[idx_a], sem)
        sa.wait()
        sb = pltpu.async_copy(ones_b, mask_hbm.at[idx_b], sem)
        sb.wait()

    return _plmpmd._mpmd_map(
        [(mesh, mask_body)],
        jax.ShapeDtypeStruct((n_words,), jnp.float32),
        input_output_aliases={1: 0},
        scratch_types=[
            pltpu.VMEM((n_a,), jnp.int32),
            pltpu.VMEM((n_b,), jnp.int32),
            pltpu.VMEM((n_a,), jnp.float32),
            pltpu.VMEM((n_b,), jnp.float32),
            pltpu.SemaphoreType.DMA,
        ],
        compiler_params=pltpu.CompilerParams(needs_layout_passes=False),
    )


def _blend_body(
    state_ref, w_ref, b_ref, x_ref, m_ref, o_ref, arfa_ref, mt_ref, ntiles
):
    @pl.when(pl.program_id(0) == 0)
    def _():
        z = lax.dot_general(
            w_ref[...],
            state_ref[...],
            (((1,), (1,)), ((), ())),
            preferred_element_type=jnp.float32,
        )
        arfa_ref[...] = jax.nn.sigmoid(z + b_ref[...])
        for ib in range(mt_ref.shape[0]):
            mt_ref[ib] = m_ref[ib * ntiles : (ib + 1) * ntiles, :].T

    i = pl.program_id(0)
    arfa = arfa_ref[...]
    mta = mt_ref[i]
    for t in range(ntiles):
        m_col = mta[:, t : t + 1] != 0.0
        sl = slice(t * 128, (t + 1) * 128)
        x_blk = x_ref[sl, :]
        o_ref[sl, :] = jnp.where(m_col, x_blk * arfa, x_blk)


def kernel(output, state, grammar, W, b):
    B, _, V = output.shape
    H = state.shape[-1]
    G = grammar.shape[0]

    vblk = 14336
    tiles_per_blk = vblk // 128
    n_blocks = -(-V // vblk)

    rows = -(-(n_blocks * tiles_per_blk) // 8) * 8
    g_pad = -(-G // (_NUM_WORKERS * _LANES)) * (_NUM_WORKERS * _LANES)

    gpad = jnp.concatenate(
        [grammar, jnp.broadcast_to(grammar[:1], (g_pad - G,))]
    )

    zeros_flat = jnp.zeros((rows * 128,), jnp.float32)
    mask = _make_mask_kernel(rows * 128, g_pad)(gpad, zeros_flat).reshape(rows, 128)

    xt = jnp.transpose(output, (1, 2, 0)).reshape(V, B)
    state2d = state.reshape(B, H)
    b2d = b.reshape(1, 1)

    import functools as _ft

    out_t = pl.pallas_call(
        _ft.partial(_blend_body, ntiles=tiles_per_blk),
        grid=(n_blocks,),
        in_specs=[
            pl.BlockSpec((B, H), lambda i: (0, 0)),
            pl.BlockSpec((1, H), lambda i: (0, 0)),
            pl.BlockSpec((1, 1), lambda i: (0, 0)),
            pl.BlockSpec((vblk, B), lambda i: (i, 0)),
            pl.BlockSpec((rows, 128), lambda i: (0, 0)),
        ],
        out_specs=pl.BlockSpec((vblk, B), lambda i: (i, 0)),
        out_shape=jax.ShapeDtypeStruct((V, B), jnp.float32),
        scratch_shapes=[
            pltpu.VMEM((1, B), jnp.float32),
            pltpu.VMEM((n_blocks, 128, tiles_per_blk), jnp.float32),
        ],
    )(state2d, W, b2d, xt, mask)

    return jnp.transpose(out_t.reshape(1, V, B), (2, 0, 1))


# speedup vs baseline: 1.3510x; 1.3510x over previous
import functools

import jax
import jax.numpy as jnp
from jax import lax
from jax.experimental import pallas as pl
from jax.experimental.pallas import tpu as pltpu
from jax.experimental.pallas import tpu_sc as plsc

_NUM_WORKERS = 32
_LANES = 16


def _make_mask_kernel(rows: int, g_rows: int):
    rows_per_w = rows // _NUM_WORKERS
    chunk = rows_per_w * 128
    mesh = plsc.VectorSubcoreMesh(core_axis_name="c", subcore_axis_name="s")

    @functools.partial(
        pl.kernel,
        mesh=mesh,
        out_type=jax.ShapeDtypeStruct((rows, 128), jnp.float32),
        scratch_types=[
            pltpu.VMEM((g_rows, 128), jnp.int32),
            pltpu.VMEM((rows_per_w, 128), jnp.float32),
            pltpu.SemaphoreType.DMA,
        ],
        compiler_params=pltpu.CompilerParams(needs_layout_passes=False),
    )
    def mask_kernel(grammar_hbm, mask_hbm, idx_v, buf_v, sem):
        c = lax.axis_index("c")
        s = lax.axis_index("s")
        wid = s * 2 + c
        base = wid * chunk

        gcopy = pltpu.async_copy(grammar_hbm, idx_v, sem)

        zeros16 = jnp.zeros((_LANES,), jnp.float32)

        def zero_body(i, carry):
            buf_v[i // 8, pl.ds((i % 8) * _LANES, _LANES)] = zeros16
            return carry

        lax.fori_loop(0, rows_per_w * 8, zero_body, 0, unroll=8)

        gcopy.wait()

        ones16 = jnp.ones((_LANES,), jnp.float32)

        def scatter_body(j, carry):
            idx = idx_v[j // 8, pl.ds((j % 8) * _LANES, _LANES)]
            m = (idx >= base) & (idx < base + chunk)
            local = jnp.where(m, idx - base, 0)
            row = lax.shift_right_logical(local, 7)
            col = lax.bitwise_and(local, 127)
            plsc.store_scatter(buf_v, [row, col], ones16, mask=m)
            return carry

        lax.fori_loop(0, g_rows * 8, scatter_body, 0, unroll=8)

        pltpu.sync_copy(buf_v, mask_hbm.at[pl.ds(wid * rows_per_w, rows_per_w), :])

    return mask_kernel


def _blend_body(
    state_ref, w_ref, b_ref, x_ref, m_ref, o_ref, arfa_ref, mt_ref, ntiles
):
    @pl.when(pl.program_id(0) == 0)
    def _():
        z = lax.dot_general(
            w_ref[...],
            state_ref[...],
            (((1,), (1,)), ((), ())),
            preferred_element_type=jnp.float32,
        )
        arfa_ref[...] = jax.nn.sigmoid(z + b_ref[...])
        for ib in range(mt_ref.shape[0]):
            mt_ref[ib] = m_ref[ib * ntiles : (ib + 1) * ntiles, :].T

    i = pl.program_id(0)
    arfa = arfa_ref[...]
    mta = mt_ref[i]
    for t in range(ntiles):
        m_col = mta[:, t : t + 1] != 0.0
        sl = slice(t * 128, (t + 1) * 128)
        x_blk = x_ref[sl, :]
        o_ref[sl, :] = jnp.where(m_col, x_blk * arfa, x_blk)


def kernel(output, state, grammar, W, b):
    B, _, V = output.shape
    H = state.shape[-1]
    G = grammar.shape[0]

    vblk = 14336
    tiles_per_blk = vblk // 128
    n_blocks = -(-V // vblk)

    rows = -(-(n_blocks * tiles_per_blk) // (_NUM_WORKERS * 8)) * (_NUM_WORKERS * 8)
    g_rows = -(-G // 128)

    gpad = jnp.concatenate(
        [grammar, jnp.full((g_rows * 128 - G,), -1, jnp.int32)]
    ).reshape(g_rows, 128)

    mask = _make_mask_kernel(rows, g_rows)(gpad)

    xt = jnp.transpose(output, (1, 2, 0)).reshape(V, B)
    state2d = state.reshape(B, H)
    b2d = b.reshape(1, 1)

    import functools as _ft

    out_t = pl.pallas_call(
        _ft.partial(_blend_body, ntiles=tiles_per_blk),
        grid=(n_blocks,),
        in_specs=[
            pl.BlockSpec((B, H), lambda i: (0, 0)),
            pl.BlockSpec((1, H), lambda i: (0, 0)),
            pl.BlockSpec((1, 1), lambda i: (0, 0)),
            pl.BlockSpec((vblk, B), lambda i: (i, 0)),
            pl.BlockSpec((rows, 128), lambda i: (0, 0)),
        ],
        out_specs=pl.BlockSpec((vblk, B), lambda i: (i, 0)),
        out_shape=jax.ShapeDtypeStruct((V, B), jnp.float32),
        scratch_shapes=[
            pltpu.VMEM((1, B), jnp.float32),
            pltpu.VMEM((n_blocks, 128, tiles_per_blk), jnp.float32),
        ],
    )(state2d, W, b2d, xt, mask)

    return jnp.transpose(out_t.reshape(1, V, B), (2, 0, 1))
